# Initial kernel scaffold; baseline (speedup 1.0000x reference)
import jax
import jax.numpy as jnp
from jax.experimental import pallas as pl


def kernel(x):
    raise NotImplementedError("write your pallas kernel here")


# speedup vs baseline: 1.0551x; 1.0551x over previous
import functools
import math

import jax
import jax.numpy as jnp
from jax.experimental import pallas as pl
from jax.experimental.pallas import tpu as pltpu


def _layernorm_tile(x_ref, o_ref, *, inv_h: float, eps: float):
    x = x_ref[...].astype(jnp.float32)
    s1 = jnp.sum(x, axis=-1, keepdims=True)
    s2 = jnp.sum(x * x, axis=-1, keepdims=True)
    mean = s1 * inv_h
    var = s2 * inv_h - mean * mean
    scale = jax.lax.rsqrt(jnp.maximum(var, 0.0) + eps)
    o_ref[...] = ((x - mean) * scale).astype(o_ref.dtype)


def kernel(x, eps: float = 1e-5):
    shape = x.shape
    hidden = int(shape[-1])
    rows = int(math.prod(shape[:-1])) if len(shape) > 1 else 1
    x2d = x.reshape(rows, hidden)
    itemsize = jnp.dtype(x.dtype).itemsize

    target_rows = max(1, (8 << 20) // (hidden * itemsize))
    block_rows = min(rows, target_rows)
    align = 8 * (4 // max(1, itemsize)) if itemsize < 4 else 8
    block_rows = max(align, (block_rows // align) * align)
    n_blocks = pl.cdiv(rows, block_rows)

    out = pl.pallas_call(
        functools.partial(_layernorm_tile, inv_h=1.0 / hidden, eps=float(eps)),
        out_shape=jax.ShapeDtypeStruct((rows, hidden), x.dtype),
        grid=(n_blocks,),
        in_specs=[pl.BlockSpec((block_rows, hidden), lambda i: (i, 0))],
        out_specs=pl.BlockSpec((block_rows, hidden), lambda i: (i, 0)),
        compiler_params=pltpu.CompilerParams(
            dimension_semantics=("parallel",),
            vmem_limit_bytes=64 << 20,
        ),
        cost_estimate=pl.CostEstimate(
            flops=6 * rows * hidden,
            transcendentals=rows,
            bytes_accessed=2 * rows * hidden * itemsize,
        ),
    )(x2d)
    return out.reshape(shape)


# speedup vs baseline: 1.0742x; 1.0181x over previous
import functools
import math

import jax
import jax.numpy as jnp
from jax.experimental import pallas as pl
from jax.experimental.pallas import tpu as pltpu


def _layernorm_tile(x_ref, o_ref, *, inv_h: float, eps: float):
    x = x_ref[...].astype(jnp.float32)
    s1 = jnp.sum(x, axis=-1, keepdims=True)
    s2 = jnp.sum(x * x, axis=-1, keepdims=True)
    mean = s1 * inv_h
    var = s2 * inv_h - mean * mean
    scale = jax.lax.rsqrt(jnp.maximum(var, 0.0) + eps)
    o_ref[...] = ((x - mean) * scale).astype(o_ref.dtype)


def kernel(x, eps: float = 1e-5):
    shape = x.shape
    hidden = int(shape[-1])
    rows = int(math.prod(shape[:-1])) if len(shape) > 1 else 1
    x2d = x.reshape(rows, hidden)
    itemsize = jnp.dtype(x.dtype).itemsize

    target_rows = max(1, (12 << 20) // (hidden * itemsize))
    block_rows = min(rows, target_rows)
    align = 8 * (4 // max(1, itemsize)) if itemsize < 4 else 8
    block_rows = max(align, (block_rows // align) * align)
    n_blocks = pl.cdiv(rows, block_rows)

    out = pl.pallas_call(
        functools.partial(_layernorm_tile, inv_h=1.0 / hidden, eps=float(eps)),
        out_shape=jax.ShapeDtypeStruct((rows, hidden), x.dtype),
        grid=(n_blocks,),
        in_specs=[pl.BlockSpec((block_rows, hidden), lambda i: (i, 0))],
        out_specs=pl.BlockSpec((block_rows, hidden), lambda i: (i, 0)),
        compiler_params=pltpu.CompilerParams(
            dimension_semantics=("parallel",),
            vmem_limit_bytes=64 << 20,
        ),
        cost_estimate=pl.CostEstimate(
            flops=6 * rows * hidden,
            transcendentals=rows,
            bytes_accessed=2 * rows * hidden * itemsize,
        ),
    )(x2d)
    return out.reshape(shape)


# speedup vs baseline: 1.0781x; 1.0036x over previous
import functools
import math

import jax
import jax.numpy as jnp
from jax.experimental import pallas as pl
from jax.experimental.pallas import tpu as pltpu


def _layernorm_tile(x_ref, o_ref, *, inv_h: float, eps: float, chunk: int):
    for i in range(x_ref.shape[0] // chunk):
        sl = pl.ds(i * chunk, chunk)
        x = x_ref[sl, :].astype(jnp.float32)
        s1 = jnp.sum(x, axis=-1, keepdims=True)
        s2 = jnp.sum(x * x, axis=-1, keepdims=True)
        mean = s1 * inv_h
        var = s2 * inv_h - mean * mean
        scale = jax.lax.rsqrt(jnp.maximum(var, 0.0) + eps)
        o_ref[sl, :] = ((x - mean) * scale).astype(o_ref.dtype)


def kernel(x, eps: float = 1e-5):
    shape = x.shape
    hidden = int(shape[-1])
    rows = int(math.prod(shape[:-1])) if len(shape) > 1 else 1
    x2d = x.reshape(rows, hidden)
    itemsize = jnp.dtype(x.dtype).itemsize

    target_rows = max(1, (14 << 20) // (hidden * itemsize))
    block_rows = min(rows, target_rows)
    chunk = 256
    block_rows = max(chunk, (block_rows // chunk) * chunk)
    n_blocks = pl.cdiv(rows, block_rows)

    out = pl.pallas_call(
        functools.partial(_layernorm_tile, inv_h=1.0 / hidden, eps=float(eps),
                          chunk=chunk),
        out_shape=jax.ShapeDtypeStruct((rows, hidden), x.dtype),
        grid=(n_blocks,),
        in_specs=[pl.BlockSpec((block_rows, hidden), lambda i: (i, 0))],
        out_specs=pl.BlockSpec((block_rows, hidden), lambda i: (i, 0)),
        compiler_params=pltpu.CompilerParams(
            dimension_semantics=("parallel",),
            vmem_limit_bytes=64 << 20,
        ),
        cost_estimate=pl.CostEstimate(
            flops=6 * rows * hidden,
            transcendentals=rows,
            bytes_accessed=2 * rows * hidden * itemsize,
        ),
    )(x2d)
    return out.reshape(shape)


# speedup vs baseline: 1.0895x; 1.0106x over previous
import functools
import math

import jax
import jax.numpy as jnp
from jax.experimental import pallas as pl
from jax.experimental.pallas import tpu as pltpu


def _layernorm_tile(x_ref, o_ref, *, inv_h: float, eps: float, chunk: int):
    for i in range(x_ref.shape[0] // chunk):
        sl = pl.ds(i * chunk, chunk)
        x = x_ref[sl, :].astype(jnp.float32)
        s1 = jnp.sum(x, axis=-1, keepdims=True)
        s2 = jnp.sum(x * x, axis=-1, keepdims=True)
        mean = s1 * inv_h
        var = s2 * inv_h - mean * mean
        scale = jax.lax.rsqrt(jnp.maximum(var, 0.0) + eps)
        o_ref[sl, :] = ((x - mean) * scale).astype(o_ref.dtype)


def kernel(x, eps: float = 1e-5):
    shape = x.shape
    hidden = int(shape[-1])
    rows = int(math.prod(shape[:-1])) if len(shape) > 1 else 1
    x2d = x.reshape(rows, hidden)
    itemsize = jnp.dtype(x.dtype).itemsize

    target_rows = max(1, (15 << 20) // (hidden * itemsize))
    block_rows = min(rows, target_rows)
    chunk = 256
    block_rows = max(chunk, (block_rows // chunk) * chunk)
    n_blocks = pl.cdiv(rows, block_rows)

    out = pl.pallas_call(
        functools.partial(_layernorm_tile, inv_h=1.0 / hidden, eps=float(eps),
                          chunk=chunk),
        out_shape=jax.ShapeDtypeStruct((rows, hidden), x.dtype),
        grid=(n_blocks,),
        in_specs=[pl.BlockSpec((block_rows, hidden), lambda i: (i, 0))],
        out_specs=pl.BlockSpec((block_rows, hidden), lambda i: (i, 0)),
        compiler_params=pltpu.CompilerParams(
            dimension_semantics=("parallel",),
            vmem_limit_bytes=64 << 20,
        ),
        cost_estimate=pl.CostEstimate(
            flops=6 * rows * hidden,
            transcendentals=rows,
            bytes_accessed=2 * rows * hidden * itemsize,
        ),
    )(x2d)
    return out.reshape(shape)


# speedup vs baseline: 1.0912x; 1.0016x over previous
import functools
import math

import jax
import jax.numpy as jnp
from jax.experimental import pallas as pl
from jax.experimental.pallas import tpu as pltpu


def _layernorm_tile(x_ref, o_ref, *, inv_h: float, eps: float, chunk: int):
    for i in range(x_ref.shape[0] // chunk):
        sl = pl.ds(i * chunk, chunk)
        x = x_ref[sl, :].astype(jnp.float32)
        s1 = jnp.sum(x, axis=-1, keepdims=True)
        s2 = jnp.sum(x * x, axis=-1, keepdims=True)
        mean = s1 * inv_h
        var = s2 * inv_h - mean * mean
        scale = jax.lax.rsqrt(jnp.maximum(var, 0.0) + eps)
        o_ref[sl, :] = ((x - mean) * scale).astype(o_ref.dtype)


def kernel(x, eps: float = 1e-5):
    shape = x.shape
    hidden = int(shape[-1])
    rows = int(math.prod(shape[:-1])) if len(shape) > 1 else 1
    x2d = x.reshape(rows, hidden)
    itemsize = jnp.dtype(x.dtype).itemsize

    target_rows = max(1, (31 << 19) // (hidden * itemsize))
    block_rows = min(rows, target_rows)
    chunk = 256
    block_rows = max(chunk, (block_rows // chunk) * chunk)
    n_blocks = pl.cdiv(rows, block_rows)

    out = pl.pallas_call(
        functools.partial(_layernorm_tile, inv_h=1.0 / hidden, eps=float(eps),
                          chunk=chunk),
        out_shape=jax.ShapeDtypeStruct((rows, hidden), x.dtype),
        grid=(n_blocks,),
        in_specs=[pl.BlockSpec((block_rows, hidden), lambda i: (i, 0))],
        out_specs=pl.BlockSpec((block_rows, hidden), lambda i: (i, 0)),
        compiler_params=pltpu.CompilerParams(
            dimension_semantics=("parallel",),
            vmem_limit_bytes=64 << 20,
        ),
        cost_estimate=pl.CostEstimate(
            flops=6 * rows * hidden,
            transcendentals=rows,
            bytes_accessed=2 * rows * hidden * itemsize,
        ),
    )(x2d)
    return out.reshape(shape)
